# baseline (speedup 1.0000x reference)
import functools

import jax
import jax.numpy as jnp
from jax import lax
from jax.experimental import pallas as pl
from jax.experimental.pallas import tpu as pltpu
from jax.experimental.pallas import tpu_sc as plsc

_N_CORES = 2
_N_SUBCORES = 16
_N_WORKERS = _N_CORES * _N_SUBCORES

_CHUNK = 128
_K = 8
_NBUF = 8
_ZC = 80


@functools.lru_cache(maxsize=None)
def _make_sc_aggregate(n_nodes, n_edges, d):
    epw = n_edges // _N_WORKERS
    n_chunks = epw // _CHUNK
    n_super = n_chunks // _K
    assert n_edges == _N_WORKERS * n_super * _K * _CHUNK
    assert n_super % 2 == 0 and n_super >= 4
    assert (2 * _K) % _NBUF == 0 and (_K * _CHUNK) % 8 == 0
    rows_per_tile = -(-n_nodes // (_N_SUBCORES * _ZC)) * _ZC
    acc_rows = rows_per_tile * _N_SUBCORES
    mesh = plsc.VectorSubcoreMesh(core_axis_name="c", subcore_axis_name="s")

    @functools.partial(
        pl.kernel,
        out_type=jax.ShapeDtypeStruct((_N_CORES, acc_rows, d), jnp.bfloat16),
        mesh=mesh,
        compiler_params=pltpu.CompilerParams(use_tc_tiling_on_sc=False),
        scratch_types=[
            pltpu.VMEM((_K * _CHUNK,), jnp.int32),
            pltpu.VMEM((_K * _CHUNK,), jnp.int32),
            pltpu.VMEM((_K * _CHUNK,), jnp.int32),
            pltpu.VMEM((_K * _CHUNK,), jnp.int32),
        ] + [pltpu.VMEM((_CHUNK, d), jnp.bfloat16)] * _NBUF
        + [pltpu.VMEM_SHARED((acc_rows, d), jnp.bfloat16)]
        + [pltpu.SemaphoreType.DMA] * (2 + 2 * _NBUF),
    )
    def sc_aggregate(x_hbm, ei_hbm, out_hbm, sa_src, sa_dst, sb_src, sb_dst,
                     *rest):
        slot_a = (sa_src, sa_dst)
        slot_b = (sb_src, sb_dst)
        bufs = rest[:_NBUF]
        acc = rest[_NBUF]
        isem_a = rest[_NBUF + 1]
        isem_b = rest[_NBUF + 2]
        gsems = rest[_NBUF + 3:2 * _NBUF + 3]
        ssems = rest[2 * _NBUF + 3:]
        c = lax.axis_index("c")
        s = lax.axis_index("s")
        wid = s * _N_CORES + c

        zero2x16 = jnp.zeros((2, 16), jnp.bfloat16)

        def zero_body(i2, carry):
            i = pl.multiple_of(i2 * 2, 2)
            for j in range(d // 16):
                bufs[0][pl.ds(i, 2), pl.ds(j * 16, 16)] = zero2x16
            return carry

        lax.fori_loop(0, _ZC // 2, zero_body, 0)
        for q in range(rows_per_tile // _ZC):
            pltpu.sync_copy(
                bufs[0].at[pl.ds(0, _ZC)],
                acc.at[pl.ds(s * rows_per_tile + q * _ZC, _ZC)])
        plsc.subcore_barrier()

        sbsz = _K * _CHUNK

        def issue_idx(sb, slot, isem):
            base = wid * epw + sb * sbsz
            pltpu.async_copy(ei_hbm.at[pl.ds(base, sbsz)], slot[0], isem)
            pltpu.async_copy(ei_hbm.at[pl.ds(n_edges + base, sbsz)], slot[1],
                             isem)

        def wait_idx(slot, isem):
            pltpu.make_async_copy(ei_hbm.at[pl.ds(0, sbsz)], slot[0],
                                  isem).wait()
            pltpu.make_async_copy(ei_hbm.at[pl.ds(0, sbsz)], slot[1],
                                  isem).wait()

        def _idx(slot, plane, row):
            return slot[plane].at[pl.ds(row * _CHUNK, _CHUNK)]

        def issue_g(slot, row, b):
            pltpu.async_copy(x_hbm.at[_idx(slot, 0, row)], bufs[b], gsems[b])

        def wait_g(slot, row, b):
            pltpu.make_async_copy(x_hbm.at[_idx(slot, 0, row)], bufs[b],
                                  gsems[b]).wait()

        def issue_s(slot, row, b):
            pltpu.async_copy(bufs[b], acc.at[_idx(slot, 1, row)], ssems[b],
                             add=True)

        def wait_s(slot, row, b):
            pltpu.make_async_copy(bufs[b], acc.at[_idx(slot, 1, row)],
                                  ssems[b]).wait()

        def slot_row(q):
            return (slot_a if q < _K else slot_b), q % _K

        pltpu.sync_copy(ei_hbm.at[pl.ds(wid * epw, sbsz)], sa_src)
        pltpu.sync_copy(ei_hbm.at[pl.ds(n_edges + wid * epw, sbsz)], sa_dst)
        issue_idx(1, slot_b, isem_b)
        for b in range(_NBUF):
            issue_g(slot_a, b, b)

        n_rounds = 2 * _K // _NBUF
        jj_wait_b = next(j for j in range(n_rounds)
                         if _NBUF * j + 2 * _NBUF - 1 >= _K)
        jj_issue_a = next(j for j in range(n_rounds)
                          if _NBUF * j + _NBUF - 1 >= _K - 1)

        def round_(t, jj, last):
            q0 = _NBUF * jj
            for b in range(_NBUF):
                sl, row = slot_row(q0 + b)
                wait_g(sl, row, b)
                issue_s(sl, row, b)
            if jj == jj_wait_b:
                wait_idx(slot_b, isem_b)
            if jj == n_rounds - 1 and not last:
                wait_idx(slot_a, isem_a)
            for b in range(_NBUF):
                q = q0 + b
                sl, row = slot_row(q)
                wait_s(sl, row, b)
                if not (last and jj == n_rounds - 1):
                    nsl, nrow = slot_row((q + _NBUF) % (2 * _K))
                    issue_g(nsl, nrow, b)
            if jj == jj_issue_a and not last:
                issue_idx(2 * t + 2, slot_a, isem_a)
            if jj == n_rounds - 1 and not last:
                issue_idx(2 * t + 3, slot_b, isem_b)

        def pair_body(t, carry):
            for jj in range(n_rounds):
                round_(t, jj, False)
            return carry

        lax.fori_loop(0, n_super // 2 - 1, pair_body, 0)
        for jj in range(n_rounds):
            round_(n_super // 2 - 1, jj, True)
        plsc.subcore_barrier()

        pltpu.sync_copy(
            acc.at[pl.ds(s * rows_per_tile, rows_per_tile)],
            out_hbm.at[c, pl.ds(s * rows_per_tile, rows_per_tile)])

    return sc_aggregate


@functools.lru_cache(maxsize=None)
def _make_sc_combine(acc_rows, d):
    rpt = acc_rows // _N_WORKERS
    assert rpt * _N_WORKERS == acc_rows and rpt % 2 == 0
    mesh = plsc.VectorSubcoreMesh(core_axis_name="c", subcore_axis_name="s")

    @functools.partial(
        pl.kernel,
        out_type=jax.ShapeDtypeStruct((acc_rows, d), jnp.bfloat16),
        mesh=mesh,
        compiler_params=pltpu.CompilerParams(use_tc_tiling_on_sc=False),
        scratch_types=[pltpu.VMEM((rpt, d), jnp.bfloat16)] * 3,
    )
    def sc_combine(p_hbm, o_hbm, b0, b1, ob):
        c = lax.axis_index("c")
        s = lax.axis_index("s")
        wid = s * _N_CORES + c
        base = wid * rpt
        pltpu.sync_copy(p_hbm.at[0, pl.ds(base, rpt)], b0)
        pltpu.sync_copy(p_hbm.at[1, pl.ds(base, rpt)], b1)

        def body(i2, carry):
            i = pl.multiple_of(i2 * 2, 2)
            for j in range(d // 16):
                cs = pl.ds(j * 16, 16)
                ob[pl.ds(i, 2), cs] = (b0[pl.ds(i, 2), cs] +
                                       b1[pl.ds(i, 2), cs])
            return carry

        lax.fori_loop(0, rpt // 2, body, 0)
        pltpu.sync_copy(ob, o_hbm.at[pl.ds(base, rpt)])

    return sc_combine


def kernel(feature_matrix, edge_index, weights1):
    x = jnp.squeeze(feature_matrix).astype(jnp.bfloat16)
    n_nodes, d = x.shape
    n_edges = edge_index.shape[1]
    acc_rows = (-(-n_nodes // (_N_SUBCORES * _ZC)) * _ZC) * _N_SUBCORES
    grain = _N_WORKERS * _K * _CHUNK * 2
    n_pad = -n_edges % grain
    n_edges_p = n_edges + n_pad
    if n_pad:
        ei = jnp.concatenate([
            edge_index[0],
            jnp.zeros((n_pad,), jnp.int32),
            edge_index[1],
            jnp.full((n_pad,), acc_rows - 1, jnp.int32),
        ])
    else:
        ei = edge_index.reshape(2 * n_edges_p)
    partials = _make_sc_aggregate(n_nodes, n_edges_p, d)(x, ei)
    summed = _make_sc_combine(acc_rows, d)(partials)
    return summed[:n_nodes].astype(jnp.float32) * weights1


# speedup vs baseline: 3.0974x; 3.0974x over previous
import functools

import jax
import jax.numpy as jnp
from jax import lax
from jax.experimental import pallas as pl
from jax.experimental.pallas import tpu as pltpu
from jax.experimental.pallas import tpu_sc as plsc

_N_CORES = 2
_N_SUBCORES = 16
_N_WORKERS = _N_CORES * _N_SUBCORES

_CHUNK = 125
_K = 10
_NBUF = 5
_ZC = 80


@functools.lru_cache(maxsize=None)
def _make_sc_aggregate(n_nodes, n_edges, d):
    epw = n_edges // _N_WORKERS
    n_chunks = epw // _CHUNK
    n_super = n_chunks // _K
    assert n_edges == _N_WORKERS * n_super * _K * _CHUNK
    assert n_super % 2 == 0 and n_super >= 4 and _K == 10
    assert (2 * _K) % _NBUF == 0
    rows_per_tile = -(-n_nodes // (_N_SUBCORES * _ZC)) * _ZC
    acc_rows = rows_per_tile * _N_SUBCORES
    mesh = plsc.VectorSubcoreMesh(core_axis_name="c", subcore_axis_name="s")

    @functools.partial(
        pl.kernel,
        out_type=jax.ShapeDtypeStruct((_N_CORES, acc_rows, d), jnp.bfloat16),
        mesh=mesh,
        compiler_params=pltpu.CompilerParams(use_tc_tiling_on_sc=False),
        scratch_types=[
            pltpu.VMEM((_K, _CHUNK), jnp.int32),
            pltpu.VMEM((_K, _CHUNK), jnp.int32),
            pltpu.VMEM((_K, _CHUNK), jnp.int32),
            pltpu.VMEM((_K, _CHUNK), jnp.int32),
        ] + [pltpu.VMEM((_CHUNK, d), jnp.bfloat16)] * _NBUF
        + [pltpu.VMEM_SHARED((acc_rows, d), jnp.bfloat16)]
        + [pltpu.SemaphoreType.DMA] * (2 + 2 * _NBUF),
    )
    def sc_aggregate(x_hbm, ei_hbm, out_hbm, sa_src, sa_dst, sb_src, sb_dst,
                     *rest):
        slot_a = (sa_src, sa_dst)
        slot_b = (sb_src, sb_dst)
        bufs = rest[:_NBUF]
        acc = rest[_NBUF]
        isem_a = rest[_NBUF + 1]
        isem_b = rest[_NBUF + 2]
        gsems = rest[_NBUF + 3:2 * _NBUF + 3]
        ssems = rest[2 * _NBUF + 3:]
        c = lax.axis_index("c")
        s = lax.axis_index("s")
        wid = s * _N_CORES + c

        zero2x16 = jnp.zeros((2, 16), jnp.bfloat16)

        def zero_body(i2, carry):
            i = pl.multiple_of(i2 * 2, 2)
            for j in range(d // 16):
                bufs[0][pl.ds(i, 2), pl.ds(j * 16, 16)] = zero2x16
            return carry

        lax.fori_loop(0, _ZC // 2, zero_body, 0)
        for q in range(rows_per_tile // _ZC):
            pltpu.sync_copy(
                bufs[0].at[pl.ds(0, _ZC)],
                acc.at[pl.ds(s * rows_per_tile + q * _ZC, _ZC)])
        plsc.subcore_barrier()

        def issue_idx(sb, slot, isem):
            pltpu.async_copy(ei_hbm.at[0, wid, pl.ds(sb * _K, _K)], slot[0],
                             isem)
            pltpu.async_copy(ei_hbm.at[1, wid, pl.ds(sb * _K, _K)], slot[1],
                             isem)

        def wait_idx(slot, isem):
            pltpu.make_async_copy(ei_hbm.at[0, wid, pl.ds(0, _K)], slot[0],
                                  isem).wait()
            pltpu.make_async_copy(ei_hbm.at[1, wid, pl.ds(0, _K)], slot[1],
                                  isem).wait()

        def issue_g(slot, row, b):
            pltpu.async_copy(x_hbm.at[slot[0].at[row]], bufs[b], gsems[b])

        def wait_g(slot, row, b):
            pltpu.make_async_copy(x_hbm.at[slot[0].at[row]], bufs[b],
                                  gsems[b]).wait()

        def issue_s(slot, row, b):
            pltpu.async_copy(bufs[b], acc.at[slot[1].at[row]], ssems[b],
                             add=True)

        def wait_s(slot, row, b):
            pltpu.make_async_copy(bufs[b], acc.at[slot[1].at[row]],
                                  ssems[b]).wait()

        def slot_row(q):
            return (slot_a if q < _K else slot_b), q % _K

        pltpu.sync_copy(ei_hbm.at[0, wid, pl.ds(0, _K)], sa_src)
        pltpu.sync_copy(ei_hbm.at[1, wid, pl.ds(0, _K)], sa_dst)
        issue_idx(1, slot_b, isem_b)
        for b in range(_NBUF):
            issue_g(slot_a, b, b)

        n_rounds = 2 * _K // _NBUF
        jj_wait_b = next(j for j in range(n_rounds)
                         if _NBUF * j + 2 * _NBUF - 1 >= _K)
        jj_issue_a = next(j for j in range(n_rounds)
                          if _NBUF * j + _NBUF - 1 >= _K - 1)

        def round_(t, jj, last):
            q0 = _NBUF * jj
            for b in range(_NBUF):
                sl, row = slot_row(q0 + b)
                wait_g(sl, row, b)
                issue_s(sl, row, b)
            if jj == jj_wait_b:
                wait_idx(slot_b, isem_b)
            if jj == n_rounds - 1 and not last:
                wait_idx(slot_a, isem_a)
            for b in range(_NBUF):
                q = q0 + b
                sl, row = slot_row(q)
                wait_s(sl, row, b)
                if not (last and jj == n_rounds - 1):
                    nsl, nrow = slot_row((q + _NBUF) % (2 * _K))
                    issue_g(nsl, nrow, b)
            if jj == jj_issue_a and not last:
                issue_idx(2 * t + 2, slot_a, isem_a)
            if jj == n_rounds - 1 and not last:
                issue_idx(2 * t + 3, slot_b, isem_b)

        def pair_body(t, carry):
            for jj in range(n_rounds):
                round_(t, jj, False)
            return carry

        lax.fori_loop(0, n_super // 2 - 1, pair_body, 0)
        for jj in range(n_rounds):
            round_(n_super // 2 - 1, jj, True)
        plsc.subcore_barrier()

        pltpu.sync_copy(
            acc.at[pl.ds(s * rows_per_tile, rows_per_tile)],
            out_hbm.at[c, pl.ds(s * rows_per_tile, rows_per_tile)])

    return sc_aggregate


@functools.lru_cache(maxsize=None)
def _make_sc_combine(acc_rows, d):
    rpt = acc_rows // _N_WORKERS
    assert rpt * _N_WORKERS == acc_rows and rpt % 2 == 0
    mesh = plsc.VectorSubcoreMesh(core_axis_name="c", subcore_axis_name="s")

    @functools.partial(
        pl.kernel,
        out_type=jax.ShapeDtypeStruct((acc_rows, d), jnp.bfloat16),
        mesh=mesh,
        compiler_params=pltpu.CompilerParams(use_tc_tiling_on_sc=False),
        scratch_types=[pltpu.VMEM((rpt, d), jnp.bfloat16)] * 3,
    )
    def sc_combine(p_hbm, o_hbm, b0, b1, ob):
        c = lax.axis_index("c")
        s = lax.axis_index("s")
        wid = s * _N_CORES + c
        base = wid * rpt
        pltpu.sync_copy(p_hbm.at[0, pl.ds(base, rpt)], b0)
        pltpu.sync_copy(p_hbm.at[1, pl.ds(base, rpt)], b1)

        def body(i2, carry):
            i = pl.multiple_of(i2 * 2, 2)
            for j in range(d // 16):
                cs = pl.ds(j * 16, 16)
                ob[pl.ds(i, 2), cs] = (b0[pl.ds(i, 2), cs] +
                                       b1[pl.ds(i, 2), cs])
            return carry

        lax.fori_loop(0, rpt // 2, body, 0)
        pltpu.sync_copy(ob, o_hbm.at[pl.ds(base, rpt)])

    return sc_combine


def kernel(feature_matrix, edge_index, weights1):
    x = jnp.squeeze(feature_matrix).astype(jnp.bfloat16)
    n_nodes, d = x.shape
    n_edges = edge_index.shape[1]
    acc_rows = (-(-n_nodes // (_N_SUBCORES * _ZC)) * _ZC) * _N_SUBCORES
    grain = _N_WORKERS * _K * _CHUNK * 2
    n_pad = -n_edges % grain
    if n_pad:
        pad = jnp.stack([
            jnp.zeros((n_pad,), jnp.int32),
            jnp.full((n_pad,), acc_rows - 1, jnp.int32),
        ])
        edge_index = jnp.concatenate([edge_index, pad], axis=1)
    n_edges_p = n_edges + n_pad
    n_chunks = n_edges_p // (_N_WORKERS * _CHUNK)
    ei = edge_index.reshape(2, _N_WORKERS, n_chunks, _CHUNK)
    partials = _make_sc_aggregate(n_nodes, n_edges_p, d)(x, ei)
    summed = _make_sc_combine(acc_rows, d)(partials)
    return summed[:n_nodes].astype(jnp.float32) * weights1


# speedup vs baseline: 3.1408x; 1.0140x over previous
import functools

import jax
import jax.numpy as jnp
from jax import lax
from jax.experimental import pallas as pl
from jax.experimental.pallas import tpu as pltpu
from jax.experimental.pallas import tpu_sc as plsc

_N_CORES = 2
_N_SUBCORES = 16
_N_WORKERS = _N_CORES * _N_SUBCORES

_CHUNK = 125
_K = 10
_NBUF = 10
_ZC = 80


@functools.lru_cache(maxsize=None)
def _make_sc_aggregate(n_nodes, n_edges, d):
    epw = n_edges // _N_WORKERS
    n_chunks = epw // _CHUNK
    n_super = n_chunks // _K
    assert n_edges == _N_WORKERS * n_super * _K * _CHUNK
    assert n_super % 2 == 0 and n_super >= 4 and _K == 10
    assert (2 * _K) % _NBUF == 0
    rows_per_tile = -(-n_nodes // (_N_SUBCORES * _ZC)) * _ZC
    acc_rows = rows_per_tile * _N_SUBCORES
    mesh = plsc.VectorSubcoreMesh(core_axis_name="c", subcore_axis_name="s")

    @functools.partial(
        pl.kernel,
        out_type=jax.ShapeDtypeStruct((_N_CORES, acc_rows, d), jnp.bfloat16),
        mesh=mesh,
        compiler_params=pltpu.CompilerParams(use_tc_tiling_on_sc=False),
        scratch_types=[
            pltpu.VMEM((_K, _CHUNK), jnp.int32),
            pltpu.VMEM((_K, _CHUNK), jnp.int32),
            pltpu.VMEM((_K, _CHUNK), jnp.int32),
            pltpu.VMEM((_K, _CHUNK), jnp.int32),
        ] + [pltpu.VMEM((_CHUNK, d), jnp.bfloat16)] * _NBUF
        + [pltpu.VMEM_SHARED((acc_rows, d), jnp.bfloat16)]
        + [pltpu.SemaphoreType.DMA] * (2 + 2 * _NBUF),
    )
    def sc_aggregate(x_hbm, ei_hbm, out_hbm, sa_src, sa_dst, sb_src, sb_dst,
                     *rest):
        slot_a = (sa_src, sa_dst)
        slot_b = (sb_src, sb_dst)
        bufs = rest[:_NBUF]
        acc = rest[_NBUF]
        isem_a = rest[_NBUF + 1]
        isem_b = rest[_NBUF + 2]
        gsems = rest[_NBUF + 3:2 * _NBUF + 3]
        ssems = rest[2 * _NBUF + 3:]
        c = lax.axis_index("c")
        s = lax.axis_index("s")
        wid = s * _N_CORES + c

        zero2x16 = jnp.zeros((2, 16), jnp.bfloat16)

        def zero_body(i2, carry):
            i = pl.multiple_of(i2 * 2, 2)
            for j in range(d // 16):
                bufs[0][pl.ds(i, 2), pl.ds(j * 16, 16)] = zero2x16
            return carry

        lax.fori_loop(0, _ZC // 2, zero_body, 0)
        for q in range(rows_per_tile // _ZC):
            pltpu.sync_copy(
                bufs[0].at[pl.ds(0, _ZC)],
                acc.at[pl.ds(s * rows_per_tile + q * _ZC, _ZC)])
        plsc.subcore_barrier()

        def issue_idx(sb, slot, isem):
            pltpu.async_copy(ei_hbm.at[0, wid, pl.ds(sb * _K, _K)], slot[0],
                             isem)
            pltpu.async_copy(ei_hbm.at[1, wid, pl.ds(sb * _K, _K)], slot[1],
                             isem)

        def wait_idx(slot, isem):
            pltpu.make_async_copy(ei_hbm.at[0, wid, pl.ds(0, _K)], slot[0],
                                  isem).wait()
            pltpu.make_async_copy(ei_hbm.at[1, wid, pl.ds(0, _K)], slot[1],
                                  isem).wait()

        def issue_g(slot, row, b):
            pltpu.async_copy(x_hbm.at[slot[0].at[row]], bufs[b], gsems[b])

        def wait_g(slot, row, b):
            pltpu.make_async_copy(x_hbm.at[slot[0].at[row]], bufs[b],
                                  gsems[b]).wait()

        def issue_s(slot, row, b):
            pltpu.async_copy(bufs[b], acc.at[slot[1].at[row]], ssems[b],
                             add=True)

        def wait_s(slot, row, b):
            pltpu.make_async_copy(bufs[b], acc.at[slot[1].at[row]],
                                  ssems[b]).wait()

        def slot_row(q):
            return (slot_a if q < _K else slot_b), q % _K

        pltpu.sync_copy(ei_hbm.at[0, wid, pl.ds(0, _K)], sa_src)
        pltpu.sync_copy(ei_hbm.at[1, wid, pl.ds(0, _K)], sa_dst)
        issue_idx(1, slot_b, isem_b)
        for b in range(_NBUF):
            issue_g(slot_a, b, b)

        n_rounds = 2 * _K // _NBUF
        jj_wait_b = next(j for j in range(n_rounds)
                         if _NBUF * j + 2 * _NBUF - 1 >= _K)
        jj_issue_a = next(j for j in range(n_rounds)
                          if _NBUF * j + _NBUF - 1 >= _K - 1)

        def round_(t, jj, last):
            q0 = _NBUF * jj
            for b in range(_NBUF):
                sl, row = slot_row(q0 + b)
                wait_g(sl, row, b)
                issue_s(sl, row, b)
            if jj == jj_wait_b:
                wait_idx(slot_b, isem_b)
            if jj == n_rounds - 1 and not last:
                wait_idx(slot_a, isem_a)
            for b in range(_NBUF):
                q = q0 + b
                sl, row = slot_row(q)
                wait_s(sl, row, b)
                if not (last and jj == n_rounds - 1):
                    nsl, nrow = slot_row((q + _NBUF) % (2 * _K))
                    issue_g(nsl, nrow, b)
            if jj == jj_issue_a and not last:
                issue_idx(2 * t + 2, slot_a, isem_a)
            if jj == n_rounds - 1 and not last:
                issue_idx(2 * t + 3, slot_b, isem_b)

        def pair_body(t, carry):
            for jj in range(n_rounds):
                round_(t, jj, False)
            return carry

        lax.fori_loop(0, n_super // 2 - 1, pair_body, 0)
        for jj in range(n_rounds):
            round_(n_super // 2 - 1, jj, True)
        plsc.subcore_barrier()

        pltpu.sync_copy(
            acc.at[pl.ds(s * rows_per_tile, rows_per_tile)],
            out_hbm.at[c, pl.ds(s * rows_per_tile, rows_per_tile)])

    return sc_aggregate


@functools.lru_cache(maxsize=None)
def _make_sc_combine(acc_rows, d):
    rpt = acc_rows // _N_WORKERS
    assert rpt * _N_WORKERS == acc_rows and rpt % 2 == 0
    mesh = plsc.VectorSubcoreMesh(core_axis_name="c", subcore_axis_name="s")

    @functools.partial(
        pl.kernel,
        out_type=jax.ShapeDtypeStruct((acc_rows, d), jnp.bfloat16),
        mesh=mesh,
        compiler_params=pltpu.CompilerParams(use_tc_tiling_on_sc=False),
        scratch_types=[pltpu.VMEM((rpt, d), jnp.bfloat16)] * 3,
    )
    def sc_combine(p_hbm, o_hbm, b0, b1, ob):
        c = lax.axis_index("c")
        s = lax.axis_index("s")
        wid = s * _N_CORES + c
        base = wid * rpt
        pltpu.sync_copy(p_hbm.at[0, pl.ds(base, rpt)], b0)
        pltpu.sync_copy(p_hbm.at[1, pl.ds(base, rpt)], b1)

        def body(i2, carry):
            i = pl.multiple_of(i2 * 2, 2)
            for j in range(d // 16):
                cs = pl.ds(j * 16, 16)
                ob[pl.ds(i, 2), cs] = (b0[pl.ds(i, 2), cs] +
                                       b1[pl.ds(i, 2), cs])
            return carry

        lax.fori_loop(0, rpt // 2, body, 0)
        pltpu.sync_copy(ob, o_hbm.at[pl.ds(base, rpt)])

    return sc_combine


def kernel(feature_matrix, edge_index, weights1):
    x = jnp.squeeze(feature_matrix).astype(jnp.bfloat16)
    n_nodes, d = x.shape
    n_edges = edge_index.shape[1]
    acc_rows = (-(-n_nodes // (_N_SUBCORES * _ZC)) * _ZC) * _N_SUBCORES
    grain = _N_WORKERS * _K * _CHUNK * 2
    n_pad = -n_edges % grain
    if n_pad:
        pad = jnp.stack([
            jnp.zeros((n_pad,), jnp.int32),
            jnp.full((n_pad,), acc_rows - 1, jnp.int32),
        ])
        edge_index = jnp.concatenate([edge_index, pad], axis=1)
    n_edges_p = n_edges + n_pad
    n_chunks = n_edges_p // (_N_WORKERS * _CHUNK)
    ei = edge_index.reshape(2, _N_WORKERS, n_chunks, _CHUNK)
    partials = _make_sc_aggregate(n_nodes, n_edges_p, d)(x, ei)
    summed = _make_sc_combine(acc_rows, d)(partials)
    return summed[:n_nodes].astype(jnp.float32) * weights1
